# baseline (speedup 1.0000x reference)
import jax
import jax.numpy as jnp
from jax.experimental import pallas as pl
from jax.experimental.pallas import tpu as pltpu

_BLOCK_ROWS = 25000


def _matmul_block(x_ref, w_ref, o_ref):
    o_ref[...] = jax.lax.dot_general(
        x_ref[...].astype(jnp.bfloat16),
        w_ref[...].astype(jnp.bfloat16),
        (((1,), (1,)), ((), ())),
        preferred_element_type=jnp.float32,
    )


def kernel(x, W):
    n, nfeat = x.shape
    nclass = W.shape[0]
    bn = _BLOCK_ROWS
    grid = (pl.cdiv(n, bn),)
    out = pl.pallas_call(
        _matmul_block,
        grid=grid,
        in_specs=[
            pl.BlockSpec((bn, nfeat), lambda i: (i, 0)),
            pl.BlockSpec((nclass, nfeat), lambda i: (0, 0)),
        ],
        out_specs=pl.BlockSpec((bn, nclass), lambda i: (i, 0)),
        out_shape=jax.ShapeDtypeStruct((n, nclass), jnp.float32),
        compiler_params=pltpu.CompilerParams(
            dimension_semantics=("arbitrary",),
        ),
    )(x, W)
    return out
